# baseline (speedup 1.0000x reference)
import functools

import jax
import jax.numpy as jnp
from jax import lax
from jax.experimental import pallas as pl
from jax.experimental.pallas import tpu as pltpu
from jax.experimental.pallas import tpu_sc as plsc

N = 320000
C_IN = 128
UNITS = 64
NKEYS = 16 ** 4

NC, NS, L = 2, 16, 16
NW = NC * NS

MM_BLK = 2560

SHARD = 1024
NOWN = NKEYS // SHARD
DUMP = SHARD
BKT = 192
BKTF = BKT + L
SENT = DUMP
SW = 176640
PPW = N // NW
KCH = 2000
EB = 512
GB = 400


def _mm_body(x_ref, bxyz_ref, w_ref, b_ref, o_ref, k_ref):
    o_ref[...] = (
        jnp.dot(x_ref[...], w_ref[...], preferred_element_type=jnp.float32)
        + b_ref[...]
    )
    c = bxyz_ref[...]
    k = c[:, 0] * 4096 + c[:, 1] * 256 + c[:, 2] * 16 + c[:, 3]
    k_ref[...] = k.reshape(-1, 1)


def _linear_and_keys(inputs, bxyz_indx, W, b):
    return pl.pallas_call(
        _mm_body,
        grid=(N // MM_BLK,),
        in_specs=[
            pl.BlockSpec((MM_BLK, C_IN), lambda i: (i, 0)),
            pl.BlockSpec((MM_BLK, 4), lambda i: (i, 0)),
            pl.BlockSpec((C_IN, UNITS), lambda i: (0, 0)),
            pl.BlockSpec((1, UNITS), lambda i: (0, 0)),
        ],
        out_specs=[
            pl.BlockSpec((MM_BLK, UNITS), lambda i: (i, 0)),
            pl.BlockSpec((MM_BLK, 1), lambda i: (i, 0)),
        ],
        out_shape=[
            jax.ShapeDtypeStruct((N, UNITS), jnp.float32),
            jax.ShapeDtypeStruct((N, 1), jnp.int32),
        ],
    )(inputs, bxyz_indx, W, b.reshape(1, UNITS))


_SC_MESH = plsc.VectorSubcoreMesh(
    core_axis_name="c", subcore_axis_name="s", num_cores=NC, num_subcores=NS
)

_SC_PARAMS = pltpu.CompilerParams(
    needs_layout_passes=False, use_tc_tiling_on_sc=False
)


@functools.partial(
    pl.kernel,
    out_type=[
        jax.ShapeDtypeStruct((2 * NOWN * SW,), jnp.int32),
        jax.ShapeDtypeStruct((2 * NOWN,), jnp.int32),
    ],
    mesh=_SC_MESH,
    compiler_params=_SC_PARAMS,
    scratch_types=[
        pltpu.VMEM((KCH,), jnp.int32),
        pltpu.VMEM((NOWN * BKTF + L,), jnp.int32),
        pltpu.VMEM((NOWN + L,), jnp.int32),
        pltpu.SMEM((NOWN,), jnp.int32),
        pltpu.SMEM((NOWN,), jnp.int32),
        pltpu.SMEM((L + 1,), jnp.int32),
        pltpu.SemaphoreType.DMA,
    ],
)
def _bin(keys_hbm, spill_hbm, counts_hbm, kbuf, stage, cvm, scnt, curs,
         flist, sem):
    cid = lax.axis_index("c")
    sid = lax.axis_index("s")
    wid = sid * NC + cid
    lane = lax.iota(jnp.int32, L)
    sentv = jnp.full((L,), SENT, jnp.int32)

    def sinit(i, carry):
        stage[pl.ds(i * L, L)] = sentv
        return carry

    lax.fori_loop(0, NOWN * BKTF // L, sinit, 0)

    def cinit(i, carry):
        scnt[i] = 0
        curs[i] = 0
        return carry

    lax.fori_loop(0, NOWN, cinit, 0)
    plsc.subcore_barrier()

    def flush_bucket(own):
        off = plsc.fetch_and_add(curs.at[own], BKTF, subcore_id=0)
        base = pl.multiple_of((cid * NOWN + own) * SW + off, 8)
        sbase = pl.multiple_of(own * BKTF, 8)
        pltpu.async_copy(stage.at[pl.ds(sbase, BKTF)],
                         spill_hbm.at[pl.ds(base, BKTF)], sem).wait()

        def refill(r, carry):
            stage[pl.ds(own * BKTF + r * L, L)] = sentv
            return carry

        lax.fori_loop(0, BKTF // L, refill, 0)
        scnt[own] = 0

    def chunk_body(ch, carry):
        gbase = pl.multiple_of(wid * PPW + ch * KCH, 8)
        pltpu.async_copy(keys_hbm.at[pl.ds(gbase, KCH)], kbuf, sem).wait()

        def vec_body(v, carry2):
            kv = kbuf[pl.ds(v * L, L)]
            own_v = lax.shift_right_logical(kv, 10)
            ent_v = ((gbase + v * L + lane) * 2048) + (kv & 1023)
            nfull = jnp.int32(0)
            for j in range(L):
                own = own_v[j]
                c = scnt[own]
                pos = jnp.where(lane == j, own * BKTF + c,
                                NOWN * BKTF + lane)
                plsc.store_scatter(stage, [pos], ent_v)
                cnew = c + 1
                trig = cnew == BKT
                flist[jnp.where(trig, nfull, L)] = own
                nfull = nfull + jnp.where(trig, 1, 0)
                scnt[own] = cnew

            def do_flush(q, carry3):
                flush_bucket(flist[q])
                return carry3

            lax.fori_loop(0, nfull, do_flush, 0)
            return carry2

        return lax.fori_loop(0, KCH // L, vec_body, carry)

    lax.fori_loop(0, PPW // KCH, chunk_body, 0)

    def tail(own_t, carry):
        c_tail = scnt[own_t]

        @pl.when(c_tail > 0)
        def _():
            flush_bucket(own_t)

        return carry

    lax.fori_loop(0, NOWN, tail, 0)

    plsc.subcore_barrier()

    @pl.when(sid == 0)
    def _():
        def wv(own, carry):
            cval = curs[own]
            pos = jnp.where(lane == 0, own, NOWN + lane)
            plsc.store_scatter(cvm, [pos], jnp.full((L,), cval, jnp.int32))
            return carry

        lax.fori_loop(0, NOWN, wv, 0)
        pltpu.async_copy(cvm.at[pl.ds(0, NOWN)],
                         counts_hbm.at[pl.ds(pl.multiple_of(cid * NOWN, 8), NOWN)],
                         sem).wait()


@functools.partial(
    pl.kernel,
    out_type=jax.ShapeDtypeStruct((NKEYS, UNITS), jnp.float32),
    mesh=_SC_MESH,
    compiler_params=_SC_PARAMS,
    scratch_types=[
        pltpu.VMEM((2 * NOWN + L,), jnp.int32),
        pltpu.VMEM((EB + L,), jnp.int32),
        pltpu.VMEM((EB,), jnp.int32),
        pltpu.VMEM((EB, UNITS), jnp.float32),
        pltpu.VMEM((SHARD + 1, UNITS), jnp.float32),
        pltpu.SemaphoreType.DMA,
    ],
)
def _scatter_max(counts_hbm, spill_hbm, x_hbm, table_hbm, cbuf, ebuf, ibuf,
                 rows, table_v, sem):
    wid = lax.axis_index("s") * NC + lax.axis_index("c")
    lane = lax.iota(jnp.int32, L)
    neg = jnp.full((L,), -jnp.inf, dtype=jnp.float32)
    pltpu.async_copy(counts_hbm, cbuf.at[pl.ds(0, 2 * NOWN)], sem).wait()

    def iinit(i, carry):
        ibuf[pl.ds(i * L, L)] = jnp.zeros((L,), jnp.int32)
        return carry

    lax.fori_loop(0, EB // L, iinit, 0)

    for oo in range(2):
        own = wid + NW * oo

        def tinit(i, carry):
            for f in range(UNITS // L):
                table_v[i, pl.ds(f * L, L)] = neg
            return carry

        lax.fori_loop(0, SHARD + 1, tinit, 0)

        for sc in range(2):
            cnt = cbuf[pl.ds(sc * NOWN + own, L)][0]
            nb = (cnt + EB - 1) // EB

            def batch(b, carry):
                base = pl.multiple_of((sc * NOWN + own) * SW + b * EB, 8)
                pltpu.async_copy(spill_hbm.at[pl.ds(base, EB)],
                                 ebuf.at[pl.ds(0, EB)], sem).wait()
                rem = jnp.minimum(cnt - b * EB, EB)

                def up(u, carry2):
                    ev = ebuf[pl.ds(u * L, L)]
                    iv = lax.shift_right_logical(ev, 11)
                    valid = (u * L + lane) < rem
                    ibuf[pl.ds(u * L, L)] = jnp.where(valid, iv, 0)
                    return carry2

                lax.fori_loop(0, EB // L, up, 0)
                pltpu.async_copy(x_hbm.at[ibuf], rows, sem).wait()

                def upd(i, carry2):
                    ent = ebuf[pl.ds(i, L)][0]
                    lk = ent & 2047
                    for f in range(UNITS // L):
                        sl = pl.ds(f * L, L)
                        table_v[lk, sl] = jnp.maximum(table_v[lk, sl],
                                                      rows[i, sl])
                    return carry2

                lax.fori_loop(0, rem, upd, 0)
                return carry

            lax.fori_loop(0, nb, batch, 0)

        pltpu.async_copy(table_v.at[pl.ds(0, SHARD)],
                         table_hbm.at[pl.ds(pl.multiple_of(own * SHARD, 8), SHARD)],
                         sem).wait()


@functools.partial(
    pl.kernel,
    out_type=jax.ShapeDtypeStruct((N, UNITS), jnp.float32),
    mesh=_SC_MESH,
    compiler_params=_SC_PARAMS,
    scratch_types=[
        pltpu.VMEM((GB,), jnp.int32),
        pltpu.VMEM((GB, UNITS), jnp.float32),
        pltpu.SemaphoreType.DMA,
    ],
)
def _gather_back(keys_hbm, table_hbm, out_hbm, gkey, grow, sem):
    wid = lax.axis_index("s") * NC + lax.axis_index("c")
    base = wid * (N // NW)

    def chunk(g, carry):
        off = base + g * GB
        pltpu.sync_copy(keys_hbm.at[pl.ds(off, GB)], gkey)
        pltpu.async_copy(table_hbm.at[gkey], grow, sem).wait()
        pltpu.sync_copy(grow, out_hbm.at[pl.ds(off, GB)])
        return carry

    lax.fori_loop(0, (N // NW) // GB, chunk, 0)


def kernel(inputs, bxyz_indx, W, b):
    x, keys2d = _linear_and_keys(inputs, bxyz_indx, W, b)
    keys = keys2d.reshape(-1)
    spill, counts = _bin(keys)
    table = _scatter_max(counts, spill, x)
    g = _gather_back(keys, table)
    return jnp.concatenate([x, g], axis=1)
